# Initial kernel scaffold; baseline (speedup 1.0000x reference)
import functools

import jax
import jax.numpy as jnp
from jax import lax
from jax.experimental import pallas as pl
from jax.experimental.pallas import tpu as pltpu
from jax.experimental.pallas import tpu_sc as plsc

_NUM_CORES = 2
_NUM_SUBCORES = 16
_NW = _NUM_CORES * _NUM_SUBCORES
_LANES = 16


def _make_sc_bag_sum(B, Lb, D, table_rows, *, cb=4, nbuf=2, interpret=False):
  assert B % _NW == 0
  bpw = B // _NW
  assert bpw % cb == 0
  nchunk = bpw // cb
  assert nchunk % nbuf == 0
  assert D % _LANES == 0
  nq = D // _LANES

  mesh = plsc.VectorSubcoreMesh(
      core_axis_name="c", subcore_axis_name="s",
      num_cores=_NUM_CORES, num_subcores=_NUM_SUBCORES)

  def body(idx_hbm, w_hbm, table_hbm, out_hbm,
           idx_v, w_v, rows_v, out_v, *sems):
    cid = lax.axis_index("c")
    sid = lax.axis_index("s")
    wid = sid * _NUM_CORES + cid
    bag0 = wid * bpw

    pltpu.sync_copy(idx_hbm.at[pl.ds(bag0, bpw)], idx_v)
    pltpu.sync_copy(w_hbm.at[pl.ds(bag0 * Lb, bpw * Lb)], w_v)

    def start(chunk, b):
      for k in range(cb):
        bag = chunk * cb + k
        pltpu.async_copy(
            table_hbm.at[idx_v.at[bag]],
            rows_v.at[b, pl.ds(k * Lb, Lb)],
            sems[b])

    def drain(b):
      pltpu.make_async_copy(
          table_hbm.at[pl.ds(0, cb * Lb)], rows_v.at[b], sems[b]).wait()

    for b in range(nbuf):
      start(b, b)

    def outer(i, carry):
      for b in range(nbuf):
        chunk = i * nbuf + b
        drain(b)
        for k in range(cb):
          woff = (chunk * cb + k) * Lb

          def row_body(j, acc, _woff=woff, _b=b, _k=k):
            wv = plsc.load_gather(
                w_v, [jnp.full((_LANES,), _woff + j, jnp.int32)])
            return tuple(
                acc[q] + wv * rows_v[_b, _k * Lb + j, pl.ds(q * _LANES, _LANES)]
                for q in range(nq))

          acc = lax.fori_loop(
              0, Lb, row_body,
              tuple(jnp.zeros((_LANES,), jnp.float32) for _ in range(nq)))
          for q in range(nq):
            out_v[k, pl.ds(q * _LANES, _LANES)] = acc[q]
        pltpu.sync_copy(out_v, out_hbm.at[pl.ds(bag0 + chunk * cb, cb)])
        nxt = chunk + nbuf

        @pl.when(nxt < nchunk)
        def _():
          start(nxt, b)
      return carry

    lax.fori_loop(0, nchunk // nbuf, outer, 0)

  fn = pl.kernel(
      body,
      out_type=jax.ShapeDtypeStruct((B, D), jnp.float32),
      mesh=mesh,
      scratch_types=[
          pltpu.VMEM((bpw, Lb), jnp.int32),
          pltpu.VMEM((bpw * Lb,), jnp.float32),
          pltpu.VMEM((nbuf, cb * Lb, D), jnp.float32),
          pltpu.VMEM((cb, D), jnp.float32),
      ] + [pltpu.SemaphoreType.DMA] * nbuf,
      interpret=interpret,
  )
  return fn


def _make_tc_mlp(B, D, E, *, blk=1024, interpret=False):
  assert B % blk == 0

  def body(x_ref, w1_ref, b1_ref, w2_ref, b2_ref, o_ref):
    x = x_ref[...]
    s = jnp.sum(x * x, axis=1, keepdims=True)
    x = x / jnp.maximum(jnp.sqrt(s), 1e-12)
    h = lax.dot_general(x, w1_ref[...], (((1,), (1,)), ((), ())),
                        preferred_element_type=jnp.float32) + b1_ref[...]
    h = jnp.where(h >= 0, h, 0.01 * h)
    o_ref[...] = lax.dot_general(h, w2_ref[...], (((1,), (1,)), ((), ())),
                                 preferred_element_type=jnp.float32) + b2_ref[...]

  grid = (B // blk,)
  return pl.pallas_call(
      body,
      grid=grid,
      in_specs=[
          pl.BlockSpec((blk, E), lambda i: (i, 0)),
          pl.BlockSpec((D, E), lambda i: (0, 0)),
          pl.BlockSpec((1, D), lambda i: (0, 0)),
          pl.BlockSpec((D, D), lambda i: (0, 0)),
          pl.BlockSpec((1, D), lambda i: (0, 0)),
      ],
      out_specs=pl.BlockSpec((blk, D), lambda i: (i, 0)),
      out_shape=jax.ShapeDtypeStruct((B, D), jnp.float32),
      interpret=interpret,
  )


@jax.jit
def kernel(indices, offsets, weights, base_emb, W1, b1, W2, b2):
  del offsets
  B = 16384
  Lb = 50
  V, E = base_emb.shape
  D = W1.shape[0]
  idx2d = indices.reshape(B, Lb)
  sc = _make_sc_bag_sum(B, Lb, E, V)
  bag_sums = sc(idx2d, weights, base_emb)
  mlp = _make_tc_mlp(B, D, E)
  return mlp(bag_sums, W1, b1.reshape(1, D), W2, b2.reshape(1, D))


# speedup vs baseline: 150.3700x; 150.3700x over previous
import functools

import jax
import jax.numpy as jnp
from jax import lax
from jax.experimental import pallas as pl
from jax.experimental.pallas import tpu as pltpu
from jax.experimental.pallas import tpu_sc as plsc

_NUM_CORES = 2
_NUM_SUBCORES = 16
_NW = _NUM_CORES * _NUM_SUBCORES
_LANES = 16


def _make_sc_bag_sum(B, Lb, D, table_rows, *, cb=2, nbuf=2, interpret=False):
  assert B % _NW == 0
  bpw = B // _NW
  assert bpw % cb == 0
  nchunk = bpw // cb
  assert nchunk % nbuf == 0
  assert D % _LANES == 0
  nq = D // _LANES

  mesh = plsc.VectorSubcoreMesh(
      core_axis_name="c", subcore_axis_name="s",
      num_cores=_NUM_CORES, num_subcores=_NUM_SUBCORES)

  def body(idx_hbm, w_hbm, table_hbm, out_hbm,
           idx_v, w_v, rows_v, out_v, *sems):
    cid = lax.axis_index("c")
    sid = lax.axis_index("s")
    wid = sid * _NUM_CORES + cid
    bag0 = wid * bpw

    pltpu.sync_copy(idx_hbm.at[pl.ds(bag0, bpw)], idx_v)
    pltpu.sync_copy(w_hbm.at[pl.ds(bag0 * Lb, bpw * Lb)],
                    w_v.at[pl.ds(0, bpw * Lb)])

    def start(chunk, b):
      for k in range(cb):
        bag = chunk * cb + k
        pltpu.async_copy(
            table_hbm.at[idx_v.at[bag]],
            rows_v.at[b, pl.ds(k * Lb, Lb)],
            sems[b])

    def drain(chunk, b):
      for k in range(cb):
        bag = chunk * cb + k
        pltpu.make_async_copy(
            table_hbm.at[idx_v.at[bag]],
            rows_v.at[b, pl.ds(k * Lb, Lb)],
            sems[b]).wait()

    for b in range(nbuf):
      start(b, b)

    ngrp = (Lb + _LANES - 1) // _LANES

    def outer(i, carry):
      for b in range(nbuf):
        chunk = i * nbuf + b
        drain(chunk, b)
        for k in range(cb):
          woff = (chunk * cb + k) * Lb
          acc = [jnp.zeros((_LANES,), jnp.float32) for _ in range(nq)]
          for g in range(ngrp):
            nrows = min(_LANES, Lb - g * _LANES)
            wvec = w_v[pl.ds(woff + g * _LANES, _LANES)]
            for j2 in range(nrows):
              wv = jnp.full((_LANES,), wvec[j2])
              r = k * Lb + g * _LANES + j2
              for q in range(nq):
                acc[q] = acc[q] + wv * rows_v[b, r, pl.ds(q * _LANES, _LANES)]
          for q in range(nq):
            out_v[k, pl.ds(q * _LANES, _LANES)] = acc[q]
        pltpu.sync_copy(out_v, out_hbm.at[pl.ds(bag0 + chunk * cb, cb)])
        nxt = chunk + nbuf

        @pl.when(nxt < nchunk)
        def _():
          start(nxt, b)
      return carry

    lax.fori_loop(0, nchunk // nbuf, outer, 0)

  fn = pl.kernel(
      body,
      out_type=jax.ShapeDtypeStruct((B, D), jnp.float32),
      mesh=mesh,
      scratch_types=[
          pltpu.VMEM((bpw, Lb), jnp.int32),
          pltpu.VMEM((bpw * Lb + _LANES,), jnp.float32),
          pltpu.VMEM((nbuf, cb * Lb, D), jnp.float32),
          pltpu.VMEM((cb, D), jnp.float32),
      ] + [pltpu.SemaphoreType.DMA] * nbuf,
      compiler_params=pltpu.CompilerParams(use_tc_tiling_on_sc=False),
      interpret=interpret,
  )
  return fn


def _make_tc_mlp(B, D, E, *, blk=1024, interpret=False):
  assert B % blk == 0

  def body(x_ref, w1_ref, b1_ref, w2_ref, b2_ref, o_ref):
    x = x_ref[...]
    s = jnp.sum(x * x, axis=1, keepdims=True)
    x = x / jnp.maximum(jnp.sqrt(s), 1e-12)
    h = lax.dot_general(x, w1_ref[...], (((1,), (1,)), ((), ())),
                        preferred_element_type=jnp.float32) + b1_ref[...]
    h = jnp.where(h >= 0, h, 0.01 * h)
    o_ref[...] = lax.dot_general(h, w2_ref[...], (((1,), (1,)), ((), ())),
                                 preferred_element_type=jnp.float32) + b2_ref[...]

  grid = (B // blk,)
  return pl.pallas_call(
      body,
      grid=grid,
      in_specs=[
          pl.BlockSpec((blk, E), lambda i: (i, 0)),
          pl.BlockSpec((D, E), lambda i: (0, 0)),
          pl.BlockSpec((1, D), lambda i: (0, 0)),
          pl.BlockSpec((D, D), lambda i: (0, 0)),
          pl.BlockSpec((1, D), lambda i: (0, 0)),
      ],
      out_specs=pl.BlockSpec((blk, D), lambda i: (i, 0)),
      out_shape=jax.ShapeDtypeStruct((B, D), jnp.float32),
      interpret=interpret,
  )


@jax.jit
def kernel(indices, offsets, weights, base_emb, W1, b1, W2, b2):
  del offsets
  B = 16384
  Lb = 50
  V, E = base_emb.shape
  D = W1.shape[0]
  idx2d = indices.reshape(B, Lb)
  sc = _make_sc_bag_sum(B, Lb, E, V)
  bag_sums = sc(idx2d, weights, base_emb)
  mlp = _make_tc_mlp(B, D, E)
  return mlp(bag_sums, W1, b1.reshape(1, D), W2, b2.reshape(1, D))
